# Initial kernel scaffold; baseline (speedup 1.0000x reference)
import jax
import jax.numpy as jnp
from jax.experimental import pallas as pl


def kernel(x, rbf, idx_i, idx_j, Wk2f, Wi, bi, Wj, bj, W1, b1, W2, b2, Wd, bd, u):
    raise NotImplementedError("write your pallas kernel here")


# speedup vs baseline: 2.5129x; 2.5129x over previous
import functools

import jax
import jax.numpy as jnp
from jax import lax
from jax.experimental import pallas as pl
from jax.experimental.pallas import tpu as pltpu
from jax.experimental.pallas import tpu_sc as plsc

N = 10000
E = 320000
K = 64
F = 128
R = 2

NC = 2
NS = 16
NW = NC * NS
EPT = E // NW
C = 80
CHUNKS = EPT // C
DRAIN_TILES = 10
DRAIN_ROWS = N // DRAIN_TILES
ZROWS = 200


def _g_body(rbf_ref, w_ref, o_ref):
    o_ref[...] = jnp.dot(rbf_ref[...], w_ref[...],
                         preferred_element_type=jnp.float32)


def _xj_body(x_ref, w_ref, b_ref, o_ref):
    o_ref[...] = jnp.dot(x_ref[...], w_ref[...],
                         preferred_element_type=jnp.float32) + b_ref[...]


def _tail_body(x_ref, p_ref, wi_ref, bi_ref, w1_ref, b1_ref, w2_ref, b2_ref,
               wd_ref, bd_ref, u_ref, o_ref):
    xb = x_ref[...]
    m = (jnp.dot(xb, wi_ref[...], preferred_element_type=jnp.float32)
         + bi_ref[...] + p_ref[0] + p_ref[1])
    for r in range(R):
        t = jnp.dot(m, w1_ref[r], preferred_element_type=jnp.float32) + b1_ref[r]
        m = m + jnp.dot(t, w2_ref[r], preferred_element_type=jnp.float32) + b2_ref[r]
    o_ref[...] = (u_ref[...] * xb
                  + jnp.dot(m, wd_ref[...], preferred_element_type=jnp.float32)
                  + bd_ref[...])


def _sc_body(g_hbm, xj_hbm, idxi_hbm, idxj_hbm, out_hbm,
             idxj_v, idxi_v, rows_v, g_v, zbuf_v, acc_sh, sem):
    core = lax.axis_index("c")
    sub = lax.axis_index("s")
    tile = core * NS + sub

    z16 = jnp.zeros((16,), jnp.float32)

    def zrow(r, carry):
        for c8 in range(F // 16):
            zbuf_v[r, pl.ds(c8 * 16, 16)] = z16
        return carry

    lax.fori_loop(0, ZROWS, zrow, 0)

    @pl.when(sub < DRAIN_TILES)
    def _zero():
        for part in range(DRAIN_ROWS // ZROWS):
            pltpu.sync_copy(
                zbuf_v,
                acc_sh.at[pl.ds(sub * DRAIN_ROWS + part * ZROWS, ZROWS)])

    plsc.subcore_barrier()

    base0 = tile * EPT

    def chunk(k, carry):
        base = pl.multiple_of(base0 + k * C, 8)
        pltpu.sync_copy(idxj_hbm.at[pl.ds(base, C)], idxj_v)
        pltpu.sync_copy(idxi_hbm.at[pl.ds(base, C)], idxi_v)
        pltpu.async_copy(xj_hbm.at[idxj_v], rows_v, sem).wait()
        pltpu.sync_copy(g_hbm.at[pl.ds(base, C)], g_v)

        def mrow(r, inner):
            for c8 in range(F // 16):
                s = pl.ds(c8 * 16, 16)
                rows_v[r, s] = rows_v[r, s] * g_v[r, s]
            return inner

        lax.fori_loop(0, C, mrow, 0)
        pltpu.sync_copy(rows_v, acc_sh.at[idxi_v], add=True)
        return carry

    lax.fori_loop(0, CHUNKS, chunk, 0)
    plsc.subcore_barrier()

    @pl.when(sub < DRAIN_TILES)
    def _drain():
        pltpu.sync_copy(
            acc_sh.at[pl.ds(sub * DRAIN_ROWS, DRAIN_ROWS)],
            out_hbm.at[core, pl.ds(sub * DRAIN_ROWS, DRAIN_ROWS)])


def kernel(x, rbf, idx_i, idx_j, Wk2f, Wi, bi, Wj, bj, W1, b1, W2, b2, Wd, bd, u):
    BE = 8000
    BN = 2000

    g = pl.pallas_call(
        _g_body,
        grid=(E // BE,),
        in_specs=[
            pl.BlockSpec((BE, K), lambda i: (i, 0)),
            pl.BlockSpec((K, F), lambda i: (0, 0)),
        ],
        out_specs=pl.BlockSpec((BE, F), lambda i: (i, 0)),
        out_shape=jax.ShapeDtypeStruct((E, F), jnp.float32),
    )(rbf, Wk2f)

    xj_all = pl.pallas_call(
        _xj_body,
        grid=(N // BN,),
        in_specs=[
            pl.BlockSpec((BN, F), lambda i: (i, 0)),
            pl.BlockSpec((F, F), lambda i: (0, 0)),
            pl.BlockSpec((1, F), lambda i: (0, 0)),
        ],
        out_specs=pl.BlockSpec((BN, F), lambda i: (i, 0)),
        out_shape=jax.ShapeDtypeStruct((N, F), jnp.float32),
    )(x, Wj, bj.reshape(1, F))

    mesh = plsc.VectorSubcoreMesh(core_axis_name="c", subcore_axis_name="s")
    partials = pl.kernel(
        _sc_body,
        out_type=jax.ShapeDtypeStruct((NC, N, F), jnp.float32),
        mesh=mesh,
        scratch_types=[
            pltpu.VMEM((C,), jnp.int32),
            pltpu.VMEM((C,), jnp.int32),
            pltpu.VMEM((C, F), jnp.float32),
            pltpu.VMEM((C, F), jnp.float32),
            pltpu.VMEM((ZROWS, F), jnp.float32),
            pltpu.VMEM_SHARED((N, F), jnp.float32),
            pltpu.SemaphoreType.DMA,
        ],
    )(g, xj_all, idx_i, idx_j)

    out = pl.pallas_call(
        _tail_body,
        grid=(N // BN,),
        in_specs=[
            pl.BlockSpec((BN, F), lambda i: (i, 0)),
            pl.BlockSpec((NC, BN, F), lambda i: (0, i, 0)),
            pl.BlockSpec((F, F), lambda i: (0, 0)),
            pl.BlockSpec((1, F), lambda i: (0, 0)),
            pl.BlockSpec((R, F, F), lambda i: (0, 0, 0)),
            pl.BlockSpec((R, 1, F), lambda i: (0, 0, 0)),
            pl.BlockSpec((R, F, F), lambda i: (0, 0, 0)),
            pl.BlockSpec((R, 1, F), lambda i: (0, 0, 0)),
            pl.BlockSpec((F, F), lambda i: (0, 0)),
            pl.BlockSpec((1, F), lambda i: (0, 0)),
            pl.BlockSpec((1, F), lambda i: (0, 0)),
        ],
        out_specs=pl.BlockSpec((BN, F), lambda i: (i, 0)),
        out_shape=jax.ShapeDtypeStruct((N, F), jnp.float32),
    )(x, partials, Wi, bi.reshape(1, F), W1, b1.reshape(R, 1, F),
      W2, b2.reshape(R, 1, F), Wd, bd.reshape(1, F), u.reshape(1, F))

    return out


# speedup vs baseline: 4.3658x; 1.7374x over previous
import jax
import jax.numpy as jnp
from jax import lax
from jax.experimental import pallas as pl
from jax.experimental.pallas import tpu as pltpu
from jax.experimental.pallas import tpu_sc as plsc

N = 10000
E = 320000
K = 64
F = 128
R = 2

NC = 2
NS = 16
NW = NC * NS
EPT = E // NW
C = 40
CHUNKS = EPT // C
DRAIN_TILES = 10
DRAIN_ROWS = N // DRAIN_TILES


def _g_body(rbf_ref, w_ref, o_ref):
    o_ref[...] = jnp.dot(rbf_ref[...], w_ref[...],
                         preferred_element_type=jnp.float32)


def _xj_body(x_ref, w_ref, b_ref, o_ref):
    o_ref[...] = jnp.dot(x_ref[...], w_ref[...],
                         preferred_element_type=jnp.float32) + b_ref[...]


def _tail_body(x_ref, p_ref, wi_ref, bi_ref, w1_ref, b1_ref, w2_ref, b2_ref,
               wd_ref, bd_ref, u_ref, o_ref):
    xb = x_ref[...]
    m = (jnp.dot(xb, wi_ref[...], preferred_element_type=jnp.float32)
         + bi_ref[...] + p_ref[0] + p_ref[1])
    for r in range(R):
        t = jnp.dot(m, w1_ref[r], preferred_element_type=jnp.float32) + b1_ref[r]
        m = m + jnp.dot(t, w2_ref[r], preferred_element_type=jnp.float32) + b2_ref[r]
    o_ref[...] = (u_ref[...] * xb
                  + jnp.dot(m, wd_ref[...], preferred_element_type=jnp.float32)
                  + bd_ref[...])


def _sc_body(g_hbm, xj_hbm, idxi_hbm, idxj_hbm, out_hbm,
             idxj_v, idxi_cb, rows_v, gbuf_v, sbuf_v, acc_sh,
             gsem0, gsem1, csem0, csem1, ssem0, ssem1, isem0, isem1):
    core = lax.axis_index("c")
    sub = lax.axis_index("s")
    tile = core * NS + sub
    sems = ((gsem0, csem0, ssem0, isem0), (gsem1, csem1, ssem1, isem1))
    base0 = tile * EPT

    z16 = jnp.zeros((16,), jnp.float32)

    def zrow(r, carry):
        for c8 in range(F // 16):
            sbuf_v[0, r, pl.ds(c8 * 16, 16)] = z16
        return carry

    lax.fori_loop(0, C, zrow, 0)

    @pl.when(sub < DRAIN_TILES)
    def _zero():
        for part in range(DRAIN_ROWS // C):
            pltpu.sync_copy(
                sbuf_v.at[0],
                acc_sh.at[pl.ds(sub * DRAIN_ROWS + part * C, C)])

    pltpu.sync_copy(idxj_hbm.at[pl.ds(pl.multiple_of(base0, 8), EPT)], idxj_v)
    plsc.subcore_barrier()

    def issue(kk, b):
        gs, cs = sems[b][0], sems[b][1]
        pltpu.async_copy(xj_hbm.at[idxj_v.at[pl.ds(kk * C, C)]],
                         rows_v.at[b], gs)
        pltpu.async_copy(g_hbm.at[pl.ds(pl.multiple_of(base0 + kk * C, 8), C)],
                         gbuf_v.at[b], cs)

    def fetch_idxi(kk, b):
        pltpu.async_copy(
            idxi_hbm.at[pl.ds(pl.multiple_of(base0 + kk * C, 8), C)],
            idxi_cb.at[b], sems[b][3])

    issue(0, 0)
    issue(1, 1)
    fetch_idxi(0, 0)
    fetch_idxi(1, 1)

    def pair(kp, carry):
        k0 = kp * 2
        for b in range(2):
            kk = k0 + b
            gs, cs, ss, isem = sems[b]
            pltpu.make_async_copy(
                xj_hbm.at[idxj_v.at[pl.ds(kk * C, C)]], rows_v.at[b],
                gs).wait()
            pltpu.make_async_copy(
                g_hbm.at[pl.ds(pl.multiple_of(base0 + kk * C, 8), C)],
                gbuf_v.at[b], cs).wait()

            @pl.when(kk >= 2)
            def _recycle():
                pltpu.make_async_copy(
                    sbuf_v.at[b], acc_sh.at[idxi_cb.at[b]], ss).wait()
                fetch_idxi(kk, b)

            def mrow(r, inner):
                for c8 in range(F // 16):
                    s = pl.ds(c8 * 16, 16)
                    sbuf_v[b, r, s] = rows_v[b, r, s] * gbuf_v[b, r, s]
                return inner

            lax.fori_loop(0, C, mrow, 0)

            @pl.when(kk + 2 < CHUNKS)
            def _prefetch():
                issue(kk + 2, b)

            pltpu.make_async_copy(
                idxi_hbm.at[pl.ds(pl.multiple_of(base0 + kk * C, 8), C)],
                idxi_cb.at[b], isem).wait()
            pltpu.async_copy(sbuf_v.at[b], acc_sh.at[idxi_cb.at[b]], ss,
                             add=True)
        return carry

    lax.fori_loop(0, CHUNKS // 2, pair, 0)
    for b in range(2):
        pltpu.make_async_copy(
            sbuf_v.at[b], acc_sh.at[idxi_cb.at[b]], sems[b][2]).wait()

    plsc.subcore_barrier()

    @pl.when(sub < DRAIN_TILES)
    def _drain():
        pltpu.sync_copy(
            acc_sh.at[pl.ds(sub * DRAIN_ROWS, DRAIN_ROWS)],
            out_hbm.at[core, pl.ds(sub * DRAIN_ROWS, DRAIN_ROWS)])


def kernel(x, rbf, idx_i, idx_j, Wk2f, Wi, bi, Wj, bj, W1, b1, W2, b2, Wd, bd, u):
    BE = 8000
    BN = 2000

    g = pl.pallas_call(
        _g_body,
        grid=(E // BE,),
        in_specs=[
            pl.BlockSpec((BE, K), lambda i: (i, 0)),
            pl.BlockSpec((K, F), lambda i: (0, 0)),
        ],
        out_specs=pl.BlockSpec((BE, F), lambda i: (i, 0)),
        out_shape=jax.ShapeDtypeStruct((E, F), jnp.float32),
    )(rbf, Wk2f)

    xj_all = pl.pallas_call(
        _xj_body,
        grid=(N // BN,),
        in_specs=[
            pl.BlockSpec((BN, F), lambda i: (i, 0)),
            pl.BlockSpec((F, F), lambda i: (0, 0)),
            pl.BlockSpec((1, F), lambda i: (0, 0)),
        ],
        out_specs=pl.BlockSpec((BN, F), lambda i: (i, 0)),
        out_shape=jax.ShapeDtypeStruct((N, F), jnp.float32),
    )(x, Wj, bj.reshape(1, F))

    mesh = plsc.VectorSubcoreMesh(core_axis_name="c", subcore_axis_name="s")
    partials = pl.kernel(
        _sc_body,
        out_type=jax.ShapeDtypeStruct((NC, N, F), jnp.float32),
        mesh=mesh,
        scratch_types=[
            pltpu.VMEM((EPT,), jnp.int32),
            pltpu.VMEM((2, C), jnp.int32),
            pltpu.VMEM((2, C, F), jnp.float32),
            pltpu.VMEM((2, C, F), jnp.float32),
            pltpu.VMEM((2, C, F), jnp.float32),
            pltpu.VMEM_SHARED((N, F), jnp.float32),
            pltpu.SemaphoreType.DMA,
            pltpu.SemaphoreType.DMA,
            pltpu.SemaphoreType.DMA,
            pltpu.SemaphoreType.DMA,
            pltpu.SemaphoreType.DMA,
            pltpu.SemaphoreType.DMA,
            pltpu.SemaphoreType.DMA,
            pltpu.SemaphoreType.DMA,
        ],
    )(g, xj_all, idx_i, idx_j)

    out = pl.pallas_call(
        _tail_body,
        grid=(N // BN,),
        in_specs=[
            pl.BlockSpec((BN, F), lambda i: (i, 0)),
            pl.BlockSpec((NC, BN, F), lambda i: (0, i, 0)),
            pl.BlockSpec((F, F), lambda i: (0, 0)),
            pl.BlockSpec((1, F), lambda i: (0, 0)),
            pl.BlockSpec((R, F, F), lambda i: (0, 0, 0)),
            pl.BlockSpec((R, 1, F), lambda i: (0, 0, 0)),
            pl.BlockSpec((R, F, F), lambda i: (0, 0, 0)),
            pl.BlockSpec((R, 1, F), lambda i: (0, 0, 0)),
            pl.BlockSpec((F, F), lambda i: (0, 0)),
            pl.BlockSpec((1, F), lambda i: (0, 0)),
            pl.BlockSpec((1, F), lambda i: (0, 0)),
        ],
        out_specs=pl.BlockSpec((BN, F), lambda i: (i, 0)),
        out_shape=jax.ShapeDtypeStruct((N, F), jnp.float32),
    )(x, partials, Wi, bi.reshape(1, F), W1, b1.reshape(R, 1, F),
      W2, b2.reshape(R, 1, F), Wd, bd.reshape(1, F), u.reshape(1, F))

    return out
